# baseline (speedup 1.0000x reference)
import functools

import jax
import jax.numpy as jnp
import numpy as np
from jax import lax
from jax.experimental import pallas as pl
from jax.experimental.pallas import tpu as pltpu
from jax.experimental.pallas import tpu_sc as plsc

_B, _L, _H, _W, _C = 2, 4, 128, 128, 256
_N = 1000
_CS = 7
_R = 2048
_P = _R * _CS * _CS
_NC, _NS, _LANES = 2, 16, 16
_NW = _NC * _NS
_CH = 32
_QPW = _P // _NW
_NCHUNK = _QPW // _CH


def _sample_params(rois):
    x1 = rois[:, 0:1]
    y1 = rois[:, 1:2]
    x2 = rois[:, 2:3]
    y2 = rois[:, 3:4]
    r = lax.broadcasted_iota(jnp.int32, (_R, 1), 0)
    b = (r >= _N).astype(jnp.int32)

    c_inv224 = jnp.float32(0.00446428591)
    c_invln2 = jnp.float32(1.44269502)
    c_127_128 = jnp.float32(0.9921875)
    c_inv128 = jnp.float32(0.0078125)
    c_127_6 = jnp.float32(21.1666679)

    area = (y2 - y1) * (x2 - x1)
    lvl = jnp.log(jnp.sqrt(jnp.maximum(area, 1e-12)) * c_inv224) * c_invln2 + 4.0
    lvl = jnp.clip(jnp.round(lvl).astype(jnp.int32), 2, 5)
    li = lvl - 2
    stride = jnp.where(li == 0, 4.0,
             jnp.where(li == 1, 8.0,
             jnp.where(li == 2, 16.0, 32.0))).astype(jnp.float32)

    ax1 = x1 / stride
    ay1 = y1 / stride
    ax2 = x2 / stride
    ay2 = y2 / stride

    c = lax.broadcasted_iota(jnp.int32, (1, _CS * _CS), 1).astype(jnp.float32)
    iy = jnp.floor(c / jnp.float32(_CS))
    jx = c - jnp.float32(_CS) * iy

    dy = (ay2 * c_inv128 - ay1 * c_inv128) * c_127_6
    dx = (ax2 * c_inv128 - ax1 * c_inv128) * c_127_6
    ys = ay1 * c_127_128 + iy * dy
    xs = ax1 * c_127_128 + jx * dx

    valid = ((ys >= 0.0) & (ys <= 127.0) &
             (xs >= 0.0) & (xs <= 127.0)).astype(jnp.float32)
    y0 = jnp.floor(ys)
    x0 = jnp.floor(xs)
    wy = ys - y0
    wx = xs - x0
    y0i = jnp.clip(y0.astype(jnp.int32), 0, _H - 1)
    x0i = jnp.clip(x0.astype(jnp.int32), 0, _W - 1)
    y1i = jnp.clip(y0i + 1, 0, _H - 1)
    x1i = jnp.clip(x0i + 1, 0, _W - 1)

    base = (b * _L + li) * (_H * _W)
    i00 = base + y0i * _W + x0i
    i01 = base + y0i * _W + x1i
    i10 = base + y1i * _W + x0i
    i11 = base + y1i * _W + x1i

    omwx = 1.0 - wx
    omwy = 1.0 - wy
    w00 = omwx * omwy * valid
    w01 = wx * omwy * valid
    w10 = omwx * wy * valid
    w11 = wx * wy * valid
    return i00, i01, i10, i11, w00, w01, w10, w11


def _tc_body(rois_ref, i00_r, i01_r, i10_r, i11_r, w00_r, w01_r, w10_r, w11_r):
    outs = _sample_params(rois_ref[...])
    for ref, val in zip((i00_r, i01_r, i10_r, i11_r, w00_r, w01_r, w10_r, w11_r),
                        outs):
        ref[...] = val


_tc_prelude = pl.pallas_call(
    _tc_body,
    out_shape=([jax.ShapeDtypeStruct((_R, _CS * _CS), jnp.int32)] * 4
               + [jax.ShapeDtypeStruct((_R, _CS * _CS), jnp.float32)] * 4),
)


def _out_row_map():
    p = np.arange(_P, dtype=np.int64)
    r = p // (_CS * _CS)
    cell = p % (_CS * _CS)
    r = np.where(r >= _B * _N, r - (_R - _B * _N), r)
    b = r // _N
    n = r % _N
    row = (b * (_CS * _CS) + cell) * _N + n
    return jnp.asarray(row.reshape(_NW, _NCHUNK, _CH), dtype=jnp.int32)


@functools.cache
def _make_sc_gather_blend():
    return functools.partial(
        pl.kernel,
        out_type=jax.ShapeDtypeStruct((_B * _CS * _CS * _N, _C), jnp.float32),
        mesh=plsc.VectorSubcoreMesh(core_axis_name="c", subcore_axis_name="s",
                                    num_cores=_NC, num_subcores=_NS),
        scratch_types=[
            pltpu.VMEM((_NCHUNK, 4 * _CH), jnp.int32),
            pltpu.VMEM((_NCHUNK, _CH), jnp.int32),
            pltpu.VMEM((2, 4 * _CH), jnp.float32),
            pltpu.VMEM((2, 4 * _CH, _C), jnp.float32),
            pltpu.VMEM((2, _CH, _C), jnp.float32),
            pltpu.SemaphoreType.DMA,
            pltpu.SemaphoreType.DMA,
            pltpu.SemaphoreType.DMA,
            pltpu.SemaphoreType.DMA,
            pltpu.SemaphoreType.DMA,
            pltpu.SemaphoreType.DMA,
        ],
    )(_sc_body)


def _sc_body(fm_hbm, idx_hbm, oidx_hbm, w_hbm, out_hbm,
             idx_v, oidx_v, w_v, rows_v, out_v,
             gsem0, gsem1, wsem0, wsem1, osem0, osem1):
    wid = lax.axis_index("s") * _NC + lax.axis_index("c")
    gsems = (gsem0, gsem1)
    wsems = (wsem0, wsem1)
    osems = (osem0, osem1)
    pltpu.sync_copy(idx_hbm.at[wid], idx_v)
    pltpu.sync_copy(oidx_hbm.at[wid], oidx_v)

    def fetch(g, slot):
        pltpu.make_async_copy(w_hbm.at[wid, g], w_v.at[slot],
                              wsems[slot]).start()
        pltpu.make_async_copy(fm_hbm.at[idx_v.at[g]], rows_v.at[slot],
                              gsems[slot]).start()

    def wait_fetch(g, slot):
        pltpu.make_async_copy(w_hbm.at[wid, g], w_v.at[slot],
                              wsems[slot]).wait()
        pltpu.make_async_copy(fm_hbm.at[idx_v.at[g]], rows_v.at[slot],
                              gsems[slot]).wait()

    def put(g, slot):
        pltpu.make_async_copy(out_v.at[slot],
                              out_hbm.at[oidx_v.at[g]],
                              osems[slot]).start()

    def wait_put(g, slot):
        pltpu.make_async_copy(out_v.at[slot],
                              out_hbm.at[oidx_v.at[g]],
                              osems[slot]).wait()

    def compute(slot):
        def grp_body(q, carry2):
            wvec = w_v[slot, pl.ds(16 * q, 16)]
            for t in range(4):
                p = 4 * q + t
                w0 = wvec[4 * t]
                w1 = wvec[4 * t + 1]
                w2 = wvec[4 * t + 2]
                w3 = wvec[4 * t + 3]
                for cc in range(_C // _LANES):
                    s = pl.ds(cc * _LANES, _LANES)
                    out_v[slot, p, s] = (
                        rows_v[slot, p, s] * w0
                        + rows_v[slot, _CH + p, s] * w1
                        + rows_v[slot, 2 * _CH + p, s] * w2
                        + rows_v[slot, 3 * _CH + p, s] * w3)
            return carry2

        lax.fori_loop(0, _CH // 4, grp_body, 0)

    fetch(0, 0)

    def body2(i, carry):
        g0 = 2 * i
        g1 = g0 + 1
        fetch(g1, 1)
        wait_fetch(g0, 0)

        @pl.when(i > 0)
        def _():
            wait_put(g0 - 2, 0)

        compute(0)
        put(g0, 0)

        @pl.when(g1 + 1 < _NCHUNK)
        def _():
            fetch(g1 + 1, 0)

        wait_fetch(g1, 1)

        @pl.when(i > 0)
        def _():
            wait_put(g1 - 2, 1)

        compute(1)
        put(g1, 1)
        return carry

    lax.fori_loop(0, _NCHUNK // 2, body2, 0)
    wait_put(_NCHUNK - 2, 0)
    wait_put(_NCHUNK - 1, 1)


def _chunkify(a00, a01, a10, a11, point_major=False):
    parts = [a.reshape(_P // _CH, _CH) for a in (a00, a01, a10, a11)]
    if point_major:
        return jnp.stack(parts, axis=2).reshape(_P // _CH, 4 * _CH)
    return jnp.stack(parts, axis=1).reshape(_P // _CH, 4 * _CH)


def kernel(feature_maps, rois):
    rois_flat = rois.reshape(_B * _N, 4)
    rois_pad = jnp.concatenate(
        [rois_flat, rois_flat[_B * _N - (_R - _B * _N):]], axis=0)
    i00, i01, i10, i11, w00, w01, w10, w11 = _tc_prelude(rois_pad)
    idx_c = _chunkify(i00, i01, i10, i11).reshape(_NW, _NCHUNK, 4 * _CH)
    w_c = _chunkify(w00, w01, w10, w11, point_major=True).reshape(
        _NW, _NCHUNK, 4 * _CH)
    fm_flat = feature_maps.reshape(_B * _L * _H * _W, _C)
    out2d = _make_sc_gather_blend()(fm_flat, idx_c, _out_row_map(), w_c)
    out = out2d.reshape(_B, _CS, _CS, _N, _C)
    return out.transpose(0, 3, 1, 2, 4)


# speedup vs baseline: 1.6641x; 1.6641x over previous
import functools

import jax
import jax.numpy as jnp
import numpy as np
from jax import lax
from jax.experimental import pallas as pl
from jax.experimental.pallas import tpu as pltpu
from jax.experimental.pallas import tpu_sc as plsc

_B, _L, _H, _W, _C = 2, 4, 128, 128, 256
_N = 1000
_CS = 7
_R = 2048
_P = _R * _CS * _CS
_NC, _NS, _LANES = 2, 16, 16
_NW = _NC * _NS
_CH = 32
_QPW = _P // _NW
_NCHUNK = _QPW // _CH


def _sample_params(rois):
    x1 = rois[:, 0:1]
    y1 = rois[:, 1:2]
    x2 = rois[:, 2:3]
    y2 = rois[:, 3:4]
    r = lax.broadcasted_iota(jnp.int32, (_R, 1), 0)
    b = (r >= _N).astype(jnp.int32)

    c_inv224 = jnp.float32(0.00446428591)
    c_invln2 = jnp.float32(1.44269502)
    c_127_128 = jnp.float32(0.9921875)
    c_inv128 = jnp.float32(0.0078125)
    c_127_6 = jnp.float32(21.1666679)

    area = (y2 - y1) * (x2 - x1)
    lvl = jnp.log(jnp.sqrt(jnp.maximum(area, 1e-12)) * c_inv224) * c_invln2 + 4.0
    lvl = jnp.clip(jnp.round(lvl).astype(jnp.int32), 2, 5)
    li = lvl - 2
    stride = jnp.where(li == 0, 4.0,
             jnp.where(li == 1, 8.0,
             jnp.where(li == 2, 16.0, 32.0))).astype(jnp.float32)

    ax1 = x1 / stride
    ay1 = y1 / stride
    ax2 = x2 / stride
    ay2 = y2 / stride

    c = lax.broadcasted_iota(jnp.int32, (1, _CS * _CS), 1).astype(jnp.float32)
    iy = jnp.floor(c / jnp.float32(_CS))
    jx = c - jnp.float32(_CS) * iy

    dy = (ay2 * c_inv128 - ay1 * c_inv128) * c_127_6
    dx = (ax2 * c_inv128 - ax1 * c_inv128) * c_127_6
    ys = ay1 * c_127_128 + iy * dy
    xs = ax1 * c_127_128 + jx * dx

    valid = ((ys >= 0.0) & (ys <= 127.0) &
             (xs >= 0.0) & (xs <= 127.0)).astype(jnp.float32)
    y0 = jnp.floor(ys)
    x0 = jnp.floor(xs)
    wy = ys - y0
    wx = xs - x0
    y0i = jnp.clip(y0.astype(jnp.int32), 0, _H - 1)
    x0i = jnp.clip(x0.astype(jnp.int32), 0, _W - 1)
    y1i = jnp.clip(y0i + 1, 0, _H - 1)
    x1i = jnp.clip(x0i + 1, 0, _W - 1)

    base = (b * _L + li) * (_H * _W)
    i00 = base + y0i * _W + x0i
    i01 = base + y0i * _W + x1i
    i10 = base + y1i * _W + x0i
    i11 = base + y1i * _W + x1i

    omwx = 1.0 - wx
    omwy = 1.0 - wy
    w00 = omwx * omwy * valid
    w01 = wx * omwy * valid
    w10 = omwx * wy * valid
    w11 = wx * wy * valid
    return i00, i01, i10, i11, w00, w01, w10, w11


def _tc_body(rois_ref, i00_r, i01_r, i10_r, i11_r, w00_r, w01_r, w10_r, w11_r):
    outs = _sample_params(rois_ref[...])
    for ref, val in zip((i00_r, i01_r, i10_r, i11_r, w00_r, w01_r, w10_r, w11_r),
                        outs):
        ref[...] = val


_tc_prelude = pl.pallas_call(
    _tc_body,
    out_shape=([jax.ShapeDtypeStruct((_R, _CS * _CS), jnp.int32)] * 4
               + [jax.ShapeDtypeStruct((_R, _CS * _CS), jnp.float32)] * 4),
)


def _out_row_map():
    p = np.arange(_P, dtype=np.int64)
    r = p // (_CS * _CS)
    cell = p % (_CS * _CS)
    r = np.where(r >= _B * _N, r - (_R - _B * _N), r)
    b = r // _N
    n = r % _N
    row = (b * (_CS * _CS) + cell) * _N + n
    return jnp.asarray(row.reshape(_NW, _NCHUNK, _CH), dtype=jnp.int32)


@functools.cache
def _make_sc_gather_blend():
    return functools.partial(
        pl.kernel,
        out_type=jax.ShapeDtypeStruct((_B * _CS * _CS * _N, _C), jnp.float32),
        mesh=plsc.VectorSubcoreMesh(core_axis_name="c", subcore_axis_name="s",
                                    num_cores=_NC, num_subcores=_NS),
        scratch_types=[
            pltpu.VMEM((_NCHUNK, 4 * _CH), jnp.int32),
            pltpu.VMEM((_NCHUNK, _CH), jnp.int32),
            pltpu.VMEM((2, _CH * 16), jnp.float32),
            pltpu.VMEM((2, 4 * _CH, _C), jnp.float32),
            pltpu.VMEM((2, _CH, _C), jnp.float32),
            pltpu.SemaphoreType.DMA,
            pltpu.SemaphoreType.DMA,
            pltpu.SemaphoreType.DMA,
            pltpu.SemaphoreType.DMA,
            pltpu.SemaphoreType.DMA,
            pltpu.SemaphoreType.DMA,
        ],
    )(_sc_body)


def _sc_body(fm_hbm, idx_hbm, oidx_hbm, w_hbm, out_hbm,
             idx_v, oidx_v, w_v, rows_v, out_v,
             gsem0, gsem1, wsem0, wsem1, osem0, osem1):
    wid = lax.axis_index("s") * _NC + lax.axis_index("c")
    gsems = (gsem0, gsem1)
    wsems = (wsem0, wsem1)
    osems = (osem0, osem1)
    pltpu.sync_copy(idx_hbm.at[wid], idx_v)
    pltpu.sync_copy(oidx_hbm.at[wid], oidx_v)

    def fetch(g, slot):
        pltpu.make_async_copy(w_hbm.at[wid, g], w_v.at[slot],
                              wsems[slot]).start()
        pltpu.make_async_copy(fm_hbm.at[idx_v.at[g]], rows_v.at[slot],
                              gsems[slot]).start()

    def wait_fetch(g, slot):
        pltpu.make_async_copy(w_hbm.at[wid, g], w_v.at[slot],
                              wsems[slot]).wait()
        pltpu.make_async_copy(fm_hbm.at[idx_v.at[g]], rows_v.at[slot],
                              gsems[slot]).wait()

    def put(g, slot):
        pltpu.make_async_copy(out_v.at[slot],
                              out_hbm.at[oidx_v.at[g]],
                              osems[slot]).start()

    def wait_put(g, slot):
        pltpu.make_async_copy(out_v.at[slot],
                              out_hbm.at[oidx_v.at[g]],
                              osems[slot]).wait()

    def compute(slot):
        def pt_body(p, carry2):
            wvec = w_v[slot, pl.ds(16 * p, 16)]
            w0 = wvec[0]
            w1 = wvec[1]
            w2 = wvec[2]
            w3 = wvec[3]
            for cc in range(_C // _LANES):
                s = pl.ds(cc * _LANES, _LANES)
                out_v[slot, p, s] = (
                    rows_v[slot, p, s] * w0
                    + rows_v[slot, _CH + p, s] * w1
                    + rows_v[slot, 2 * _CH + p, s] * w2
                    + rows_v[slot, 3 * _CH + p, s] * w3)
            return carry2

        lax.fori_loop(0, _CH, pt_body, 0)

    fetch(0, 0)

    def body2(i, carry):
        g0 = 2 * i
        g1 = g0 + 1
        fetch(g1, 1)
        wait_fetch(g0, 0)

        @pl.when(i > 0)
        def _():
            wait_put(g0 - 2, 0)

        compute(0)
        put(g0, 0)

        @pl.when(g1 + 1 < _NCHUNK)
        def _():
            fetch(g1 + 1, 0)

        wait_fetch(g1, 1)

        @pl.when(i > 0)
        def _():
            wait_put(g1 - 2, 1)

        compute(1)
        put(g1, 1)
        return carry

    lax.fori_loop(0, _NCHUNK // 2, body2, 0)
    wait_put(_NCHUNK - 2, 0)
    wait_put(_NCHUNK - 1, 1)


def _chunkify(a00, a01, a10, a11, point_major=False):
    parts = [a.reshape(_P // _CH, _CH) for a in (a00, a01, a10, a11)]
    if point_major:
        out = jnp.stack(parts, axis=2)
        return jnp.pad(out, ((0, 0), (0, 0), (0, 12)))
    return jnp.stack(parts, axis=1).reshape(_P // _CH, 4 * _CH)


def kernel(feature_maps, rois):
    rois_flat = rois.reshape(_B * _N, 4)
    rois_pad = jnp.concatenate(
        [rois_flat, rois_flat[_B * _N - (_R - _B * _N):]], axis=0)
    i00, i01, i10, i11, w00, w01, w10, w11 = _tc_prelude(rois_pad)
    idx_c = _chunkify(i00, i01, i10, i11).reshape(_NW, _NCHUNK, 4 * _CH)
    w_c = _chunkify(w00, w01, w10, w11, point_major=True).reshape(
        _NW, _NCHUNK, _CH * 16)
    fm_flat = feature_maps.reshape(_B * _L * _H * _W, _C)
    out2d = _make_sc_gather_blend()(fm_flat, idx_c, _out_row_map(), w_c)
    out = out2d.reshape(_B, _CS, _CS, _N, _C)
    return out.transpose(0, 3, 1, 2, 4)
